# baseline (speedup 1.0000x reference)
import functools

import jax
import jax.numpy as jnp
from jax import lax
from jax.experimental import pallas as pl
from jax.experimental.pallas import tpu as pltpu
from jax.experimental.pallas import tpu_sc as plsc

KMER_NUM = 1000000
DIM = 2
LATENT_DIM = 7
BATCH = 16384

NUM_CORES = 2
NUM_SUBCORES = 16
NUM_WORKERS = NUM_CORES * NUM_SUBCORES
N_IDX = 2 * BATCH
B_PER_W = N_IDX // NUM_WORKERS


def _sc_gather(embs, flat_idx):
    mesh = plsc.VectorSubcoreMesh(core_axis_name="c", subcore_axis_name="s")

    @functools.partial(
        pl.kernel,
        mesh=mesh,
        compiler_params=pltpu.CompilerParams(use_tc_tiling_on_sc=False),
        out_type=jax.ShapeDtypeStruct((N_IDX, LATENT_DIM), jnp.float32),
        scratch_types=[
            pltpu.VMEM((B_PER_W,), jnp.int32),
            pltpu.VMEM((B_PER_W, LATENT_DIM), jnp.float32),
            pltpu.SemaphoreType.DMA,
        ],
    )
    def k(table_hbm, idx_hbm, out_hbm, idx_v, rows_v, sem):
        wid = lax.axis_index("s") * NUM_CORES + lax.axis_index("c")
        base = wid * B_PER_W
        pltpu.sync_copy(idx_hbm.at[pl.ds(base, B_PER_W)], idx_v)
        pltpu.async_copy(table_hbm.at[idx_v], rows_v, sem).wait()
        pltpu.sync_copy(rows_v, out_hbm.at[pl.ds(base, B_PER_W)])

    return k(embs, flat_idx)


def _softmax0(e):
    m = jnp.max(e, axis=0, keepdims=True)
    ex = jnp.exp(e - m)
    return ex / jnp.sum(ex, axis=0, keepdims=True)


def _tc_body(a_ref, e0_ref, e1_ref, deg_ref, out_ref):
    a = _softmax0(_softmax0(a_ref[...]))
    p0 = _softmax0(e0_ref[...])
    p1 = _softmax0(e1_ref[...])
    q = p0 - p1
    d0 = jnp.sum(q * a[:, 0:1], axis=0, keepdims=True)
    d1 = jnp.sum(q * a[:, 1:2], axis=0, keepdims=True)
    dist = jnp.abs(d0) + jnp.abs(d1)
    contrib = deg_ref[...] * dist + jnp.exp(-dist)
    out_ref[0, 0] = jnp.sum(contrib)


def _tc_epilogue(gT, degrees, A):
    out = pl.pallas_call(
        _tc_body,
        grid=(1,),
        in_specs=[
            pl.BlockSpec((LATENT_DIM, DIM), lambda i: (0, 0)),
            pl.BlockSpec((LATENT_DIM, BATCH), lambda i: (0, 0)),
            pl.BlockSpec((LATENT_DIM, BATCH), lambda i: (0, 1)),
            pl.BlockSpec((1, BATCH), lambda i: (0, 0)),
        ],
        out_specs=pl.BlockSpec(memory_space=pltpu.SMEM),
        out_shape=jax.ShapeDtypeStruct((1, 1), jnp.float32),
    )(A, gT, gT, degrees.reshape(1, BATCH))
    return out.reshape(())


def kernel(x, degrees, A, embs):
    flat_idx = x.T.reshape(N_IDX)
    g = _sc_gather(embs, flat_idx)
    gT = g.T
    return _tc_epilogue(gT, degrees, A)


# speedup vs baseline: 10.7149x; 10.7149x over previous
import functools

import jax
import jax.numpy as jnp
from jax import lax
from jax.experimental import pallas as pl
from jax.experimental.pallas import tpu as pltpu
from jax.experimental.pallas import tpu_sc as plsc

KMER_NUM = 1000000
DIM = 2
LATENT_DIM = 7
BATCH = 16384

NUM_CORES = 2
NUM_SUBCORES = 16
NUM_WORKERS = NUM_CORES * NUM_SUBCORES
PAIRS_PER_W = BATCH // NUM_WORKERS
LANES = 16
CHUNKS = PAIRS_PER_W // LANES

TC_BLOCK = 131072
TC_GRID = -(-KMER_NUM // TC_BLOCK)


def _tc_project_body(a_ref, et_ref, f0_ref, f1_ref):
    a = a_ref[...]
    for _ in range(2):
        a = jnp.exp(a - jnp.max(a, axis=0, keepdims=True))
        a = a / jnp.sum(a, axis=0, keepdims=True)
    e = et_ref[...]
    ex = jnp.exp(e - jnp.max(e, axis=0, keepdims=True))
    p = ex / jnp.sum(ex, axis=0, keepdims=True)
    f0 = jnp.sum(p * a[:, 0:1], axis=0, keepdims=True)
    f1 = jnp.sum(p * a[:, 1:2], axis=0, keepdims=True)
    f0_ref[...] = f0[0]
    f1_ref[...] = f1[0]


def _tc_project(A, embsT):
    return pl.pallas_call(
        _tc_project_body,
        grid=(TC_GRID,),
        in_specs=[
            pl.BlockSpec((LATENT_DIM, DIM), lambda i: (0, 0)),
            pl.BlockSpec((LATENT_DIM, TC_BLOCK), lambda i: (0, i)),
        ],
        out_specs=[
            pl.BlockSpec((TC_BLOCK,), lambda i: (i,)),
            pl.BlockSpec((TC_BLOCK,), lambda i: (i,)),
        ],
        out_shape=[
            jax.ShapeDtypeStruct((KMER_NUM,), jnp.float32),
            jax.ShapeDtypeStruct((KMER_NUM,), jnp.float32),
        ],
    )(A, embsT)


def _sc_pair_loss(f0, f1, idx0, idx1, degrees):
    mesh = plsc.VectorSubcoreMesh(core_axis_name="c", subcore_axis_name="s")

    @functools.partial(
        pl.kernel,
        mesh=mesh,
        compiler_params=pltpu.CompilerParams(use_tc_tiling_on_sc=False),
        out_type=jax.ShapeDtypeStruct((NUM_WORKERS, LANES), jnp.float32),
        scratch_types=[
            pltpu.VMEM((PAIRS_PER_W,), jnp.int32),
            pltpu.VMEM((PAIRS_PER_W,), jnp.int32),
            pltpu.VMEM((PAIRS_PER_W,), jnp.float32),
            pltpu.VMEM((PAIRS_PER_W,), jnp.float32),
            pltpu.VMEM((PAIRS_PER_W,), jnp.float32),
            pltpu.VMEM((PAIRS_PER_W,), jnp.float32),
            pltpu.VMEM((PAIRS_PER_W,), jnp.float32),
            pltpu.VMEM((LANES,), jnp.float32),
            pltpu.SemaphoreType.DMA,
            pltpu.SemaphoreType.DMA,
            pltpu.SemaphoreType.DMA,
            pltpu.SemaphoreType.DMA,
        ],
    )
    def k(f0_hbm, f1_hbm, i0_hbm, i1_hbm, deg_hbm, out_hbm,
          i0_v, i1_v, a0_v, a1_v, b0_v, b1_v, deg_v, acc_v,
          sem0, sem1, sem2, sem3):
        wid = lax.axis_index("s") * NUM_CORES + lax.axis_index("c")
        base = wid * PAIRS_PER_W
        pltpu.sync_copy(i0_hbm.at[pl.ds(base, PAIRS_PER_W)], i0_v)
        pltpu.sync_copy(i1_hbm.at[pl.ds(base, PAIRS_PER_W)], i1_v)
        c0 = pltpu.async_copy(f0_hbm.at[i0_v], a0_v, sem0)
        c1 = pltpu.async_copy(f0_hbm.at[i1_v], a1_v, sem1)
        c2 = pltpu.async_copy(f1_hbm.at[i0_v], b0_v, sem2)
        c3 = pltpu.async_copy(f1_hbm.at[i1_v], b1_v, sem3)
        pltpu.sync_copy(deg_hbm.at[pl.ds(base, PAIRS_PER_W)], deg_v)
        c0.wait()
        c1.wait()
        c2.wait()
        c3.wait()

        def body(i, acc):
            s = pl.ds(i * LANES, LANES)
            d = jnp.abs(a0_v[s] - a1_v[s]) + jnp.abs(b0_v[s] - b1_v[s])
            return acc + deg_v[s] * d + jnp.exp(-d)

        acc = lax.fori_loop(0, CHUNKS, body, jnp.zeros((LANES,), jnp.float32))
        acc_v[...] = acc
        pltpu.sync_copy(acc_v, out_hbm.at[wid])

    return k(f0, f1, idx0, idx1, degrees)


def kernel(x, degrees, A, embs):
    f0, f1 = _tc_project(A, embs.T)
    partials = _sc_pair_loss(f0, f1, x[:, 0], x[:, 1], degrees)
    return jnp.sum(partials)


# speedup vs baseline: 17.1715x; 1.6026x over previous
import functools

import jax
import jax.numpy as jnp
from jax import lax
from jax.experimental import pallas as pl
from jax.experimental.pallas import tpu as pltpu
from jax.experimental.pallas import tpu_sc as plsc

KMER_NUM = 1000000
DIM = 2
LATENT_DIM = 7
BATCH = 16384

NUM_CORES = 2
NUM_SUBCORES = 16
NUM_WORKERS = NUM_CORES * NUM_SUBCORES
PAIRS_PER_W = BATCH // NUM_WORKERS
LANES = 16
CHUNKS = PAIRS_PER_W // LANES

TC_BLOCK = 131072
TC_GRID = -(-KMER_NUM // TC_BLOCK)


def _tc_project_body(a_ref, et_ref, f0_ref, f1_ref):
    a = a_ref[...]
    for _ in range(2):
        a = jnp.exp(a - jnp.max(a, axis=0, keepdims=True))
        a = a / jnp.sum(a, axis=0, keepdims=True)
    e = et_ref[...]
    ex = jnp.exp(e)
    ex8 = jnp.concatenate([ex, jnp.zeros((1, TC_BLOCK), jnp.float32)], axis=0)
    m = jnp.concatenate([a, jnp.ones((LATENT_DIM, 1), jnp.float32)], axis=1)
    m8 = jnp.concatenate([m, jnp.zeros((1, 3), jnp.float32)], axis=0)
    f3 = jax.lax.dot_general(
        m8.astype(jnp.bfloat16),
        ex8.astype(jnp.bfloat16),
        (((0,), (0,)), ((), ())),
        preferred_element_type=jnp.float32,
    )
    r = 1.0 / f3[2:3]
    f0_ref[...] = (f3[0:1] * r)[0]
    f1_ref[...] = (f3[1:2] * r)[0]


def _tc_project(A, embsT):
    return pl.pallas_call(
        _tc_project_body,
        grid=(TC_GRID,),
        in_specs=[
            pl.BlockSpec((LATENT_DIM, DIM), lambda i: (0, 0)),
            pl.BlockSpec((LATENT_DIM, TC_BLOCK), lambda i: (0, i)),
        ],
        out_specs=[
            pl.BlockSpec((TC_BLOCK,), lambda i: (i,)),
            pl.BlockSpec((TC_BLOCK,), lambda i: (i,)),
        ],
        out_shape=[
            jax.ShapeDtypeStruct((KMER_NUM,), jnp.float32),
            jax.ShapeDtypeStruct((KMER_NUM,), jnp.float32),
        ],
    )(A, embsT)


def _sc_pair_loss(f0, f1, idx0, idx1, degrees):
    mesh = plsc.VectorSubcoreMesh(core_axis_name="c", subcore_axis_name="s")

    @functools.partial(
        pl.kernel,
        mesh=mesh,
        compiler_params=pltpu.CompilerParams(use_tc_tiling_on_sc=False),
        out_type=jax.ShapeDtypeStruct((NUM_WORKERS, LANES), jnp.float32),
        scratch_types=[
            pltpu.VMEM((PAIRS_PER_W,), jnp.int32),
            pltpu.VMEM((PAIRS_PER_W,), jnp.int32),
            pltpu.VMEM((PAIRS_PER_W,), jnp.float32),
            pltpu.VMEM((PAIRS_PER_W,), jnp.float32),
            pltpu.VMEM((PAIRS_PER_W,), jnp.float32),
            pltpu.VMEM((PAIRS_PER_W,), jnp.float32),
            pltpu.VMEM((PAIRS_PER_W,), jnp.float32),
            pltpu.VMEM((LANES,), jnp.float32),
            pltpu.SemaphoreType.DMA,
            pltpu.SemaphoreType.DMA,
            pltpu.SemaphoreType.DMA,
            pltpu.SemaphoreType.DMA,
        ],
    )
    def k(f0_hbm, f1_hbm, i0_hbm, i1_hbm, deg_hbm, out_hbm,
          i0_v, i1_v, a0_v, a1_v, b0_v, b1_v, deg_v, acc_v,
          sem0, sem1, sem2, sem3):
        wid = lax.axis_index("s") * NUM_CORES + lax.axis_index("c")
        base = wid * PAIRS_PER_W
        pltpu.sync_copy(i0_hbm.at[pl.ds(base, PAIRS_PER_W)], i0_v)
        pltpu.sync_copy(i1_hbm.at[pl.ds(base, PAIRS_PER_W)], i1_v)
        c0 = pltpu.async_copy(f0_hbm.at[i0_v], a0_v, sem0)
        c1 = pltpu.async_copy(f0_hbm.at[i1_v], a1_v, sem1)
        c2 = pltpu.async_copy(f1_hbm.at[i0_v], b0_v, sem2)
        c3 = pltpu.async_copy(f1_hbm.at[i1_v], b1_v, sem3)
        pltpu.sync_copy(deg_hbm.at[pl.ds(base, PAIRS_PER_W)], deg_v)
        c0.wait()
        c1.wait()
        c2.wait()
        c3.wait()

        def body(i, acc):
            s = pl.ds(i * LANES, LANES)
            d = jnp.abs(a0_v[s] - a1_v[s]) + jnp.abs(b0_v[s] - b1_v[s])
            return acc + deg_v[s] * d + jnp.exp(-d)

        acc = lax.fori_loop(0, CHUNKS, body, jnp.zeros((LANES,), jnp.float32))
        acc_v[...] = acc
        pltpu.sync_copy(acc_v, out_hbm.at[wid])

    return k(f0, f1, idx0, idx1, degrees)


def kernel(x, degrees, A, embs):
    f0, f1 = _tc_project(A, embs.T)
    partials = _sc_pair_loss(f0, f1, x[:, 0], x[:, 1], degrees)
    return jnp.sum(partials)


# speedup vs baseline: 19.8995x; 1.1589x over previous
import functools

import jax
import jax.numpy as jnp
from jax import lax
from jax.experimental import pallas as pl
from jax.experimental.pallas import tpu as pltpu
from jax.experimental.pallas import tpu_sc as plsc

KMER_NUM = 1000000
DIM = 2
LATENT_DIM = 7
BATCH = 16384

NUM_CORES = 2
NUM_SUBCORES = 16
NUM_WORKERS = NUM_CORES * NUM_SUBCORES
PAIRS_PER_W = BATCH // NUM_WORKERS
LANES = 16
CHUNKS = PAIRS_PER_W // LANES

TC_BLOCK = 262144
TC_GRID = -(-KMER_NUM // TC_BLOCK)
F_ROWS = TC_GRID * TC_BLOCK // 128

LOG2E = 1.4426950408889634


def _tc_project_body(at_ref, et_ref, f0_ref, f1_ref):
    at = at_ref[...]
    for _ in range(2):
        at = jnp.exp(at - jnp.max(at, axis=1, keepdims=True))
        at = at / jnp.sum(at, axis=1, keepdims=True)
    e = et_ref[...]
    ex = jnp.exp2(e * LOG2E)
    ex8 = jnp.concatenate([ex, jnp.zeros((1, TC_BLOCK), jnp.float32)], axis=0)
    mt = jnp.concatenate([at, jnp.ones((1, LATENT_DIM), jnp.float32)], axis=0)
    mt8 = jnp.concatenate([mt, jnp.zeros((3, 1), jnp.float32)], axis=1)
    f3 = jax.lax.dot_general(
        mt8.astype(jnp.bfloat16),
        ex8.astype(jnp.bfloat16),
        (((1,), (0,)), ((), ())),
        preferred_element_type=jnp.float32,
    )
    u0 = f3[0:1].reshape(TC_BLOCK // 128, 128)
    u1 = f3[1:2].reshape(TC_BLOCK // 128, 128)
    s = f3[2:3].reshape(TC_BLOCK // 128, 128)
    r = 1.0 / s
    f0_ref[...] = u0 * r
    f1_ref[...] = u1 * r


def _tc_project(At, embsT):
    return pl.pallas_call(
        _tc_project_body,
        grid=(TC_GRID,),
        in_specs=[
            pl.BlockSpec((DIM, LATENT_DIM), lambda i: (0, 0)),
            pl.BlockSpec((LATENT_DIM, TC_BLOCK), lambda i: (0, i)),
        ],
        out_specs=[
            pl.BlockSpec((TC_BLOCK // 128, 128), lambda i: (i, 0)),
            pl.BlockSpec((TC_BLOCK // 128, 128), lambda i: (i, 0)),
        ],
        out_shape=[
            jax.ShapeDtypeStruct((F_ROWS, 128), jnp.float32),
            jax.ShapeDtypeStruct((F_ROWS, 128), jnp.float32),
        ],
    )(At, embsT)


def _sc_pair_loss(f0, f1, xt, degrees):
    mesh = plsc.VectorSubcoreMesh(core_axis_name="c", subcore_axis_name="s")

    @functools.partial(
        pl.kernel,
        mesh=mesh,
        compiler_params=pltpu.CompilerParams(use_tc_tiling_on_sc=False),
        out_type=jax.ShapeDtypeStruct((NUM_WORKERS, LANES), jnp.float32),
        scratch_types=[
            pltpu.VMEM((PAIRS_PER_W,), jnp.int32),
            pltpu.VMEM((PAIRS_PER_W,), jnp.int32),
            pltpu.VMEM((PAIRS_PER_W,), jnp.float32),
            pltpu.VMEM((PAIRS_PER_W,), jnp.float32),
            pltpu.VMEM((PAIRS_PER_W,), jnp.float32),
            pltpu.VMEM((PAIRS_PER_W,), jnp.float32),
            pltpu.VMEM((PAIRS_PER_W,), jnp.float32),
            pltpu.VMEM((LANES,), jnp.float32),
            pltpu.SemaphoreType.DMA,
            pltpu.SemaphoreType.DMA,
            pltpu.SemaphoreType.DMA,
            pltpu.SemaphoreType.DMA,
        ],
    )
    def k(f0_hbm, f1_hbm, xt_hbm, deg_hbm, out_hbm,
          i0_v, i1_v, a0_v, a1_v, b0_v, b1_v, deg_v, acc_v,
          sem0, sem1, sem2, sem3):
        wid = lax.axis_index("s") * NUM_CORES + lax.axis_index("c")
        base = wid * PAIRS_PER_W
        pltpu.sync_copy(xt_hbm.at[0, pl.ds(base, PAIRS_PER_W)], i0_v)
        pltpu.sync_copy(xt_hbm.at[1, pl.ds(base, PAIRS_PER_W)], i1_v)
        c0 = pltpu.async_copy(f0_hbm.at[i0_v], a0_v, sem0)
        c1 = pltpu.async_copy(f0_hbm.at[i1_v], a1_v, sem1)
        c2 = pltpu.async_copy(f1_hbm.at[i0_v], b0_v, sem2)
        c3 = pltpu.async_copy(f1_hbm.at[i1_v], b1_v, sem3)
        pltpu.sync_copy(deg_hbm.at[pl.ds(base, PAIRS_PER_W)], deg_v)
        c0.wait()
        c1.wait()
        c2.wait()
        c3.wait()

        def body(i, acc):
            s = pl.ds(i * LANES, LANES)
            d = jnp.abs(a0_v[s] - a1_v[s]) + jnp.abs(b0_v[s] - b1_v[s])
            return acc + deg_v[s] * d + jnp.exp(-d)

        acc = lax.fori_loop(0, CHUNKS, body, jnp.zeros((LANES,), jnp.float32))
        acc_v[...] = acc
        pltpu.sync_copy(acc_v, out_hbm.at[wid])

    return k(f0, f1, xt, degrees)


def kernel(x, degrees, A, embs):
    f0, f1 = _tc_project(A.T, embs.T)
    partials = _sc_pair_loss(f0.reshape(-1), f1.reshape(-1), x.T, degrees)
    return jnp.sum(partials)
